# baseline (speedup 1.0000x reference)
import functools

import jax
import jax.numpy as jnp
from jax import lax
from jax.experimental import pallas as pl
from jax.experimental.pallas import tpu as pltpu
from jax.experimental.pallas import tpu_sc as plsc

BATCH = 16384
EMB_DIM = 32
NUM_CORES = 2
NUM_SUBCORES = 16
NUM_WORKERS = NUM_CORES * NUM_SUBCORES
B_PER_W = BATCH // NUM_WORKERS
CHUNK = 128
N_CHUNKS = B_PER_W // CHUNK
GROUPS = B_PER_W // 16


def _sc_body(uids_ref, gids_ref, user_ref, item_ref, out_ref,
             uid_v, gid_v, urows, grows, out_v, usem, gsem):
    wid = lax.axis_index("s") * NUM_CORES + lax.axis_index("c")
    base = wid * B_PER_W

    pltpu.sync_copy(uids_ref.at[pl.ds(wid * N_CHUNKS, N_CHUNKS)], uid_v)
    pltpu.sync_copy(gids_ref.at[pl.ds(wid * N_CHUNKS, N_CHUNKS)], gid_v)

    copies = []
    for j in range(N_CHUNKS):
        copies.append(pltpu.async_copy(
            user_ref.at[uid_v.at[j]], urows.at[pl.ds(j * CHUNK, CHUNK)], usem))
        copies.append(pltpu.async_copy(
            item_ref.at[gid_v.at[j]], grows.at[pl.ds(j * CHUNK, CHUNK)], gsem))
    for c in copies:
        c.wait()

    iota = lax.iota(jnp.int32, 16)

    def group(g, carry):
        r = g * 16
        rows = r + iota
        acc = jnp.zeros((16,), jnp.float32)
        for d in range(EMB_DIM):
            col = jnp.full((16,), d, jnp.int32)
            uv = plsc.load_gather(urows, [rows, col])
            gv = plsc.load_gather(grows, [rows, col])
            acc = acc + uv * gv
        out_v[pl.ds(r, 16)] = acc
        return carry

    lax.fori_loop(0, GROUPS, group, 0)

    pltpu.sync_copy(out_v, out_ref.at[pl.ds(base, B_PER_W)])


@jax.jit
def kernel(uids, gids, user_emb, item_emb):
    uids2d = uids.astype(jnp.int32).reshape(BATCH // CHUNK, CHUNK)
    gids2d = gids.astype(jnp.int32).reshape(BATCH // CHUNK, CHUNK)
    mesh = plsc.VectorSubcoreMesh(core_axis_name="c", subcore_axis_name="s",
                                  num_cores=NUM_CORES, num_subcores=NUM_SUBCORES)
    run = functools.partial(
        pl.kernel,
        out_type=jax.ShapeDtypeStruct((BATCH,), jnp.float32),
        mesh=mesh,
        compiler_params=pltpu.CompilerParams(
            needs_layout_passes=False, use_tc_tiling_on_sc=False),
        scratch_types=[
            pltpu.VMEM((N_CHUNKS, CHUNK), jnp.int32),
            pltpu.VMEM((N_CHUNKS, CHUNK), jnp.int32),
            pltpu.VMEM((B_PER_W, EMB_DIM), jnp.float32),
            pltpu.VMEM((B_PER_W, EMB_DIM), jnp.float32),
            pltpu.VMEM((B_PER_W,), jnp.float32),
            pltpu.SemaphoreType.DMA,
            pltpu.SemaphoreType.DMA,
        ],
    )(_sc_body)
    return run(uids2d, gids2d, user_emb, item_emb)
